# baseline (speedup 1.0000x reference)
import jax
import jax.numpy as jnp
from jax import lax
from jax.experimental import pallas as pl
from jax.experimental.pallas import tpu as pltpu
from jax.experimental.pallas import tpu_sc as plsc

N = 100000
E = 1600000
NP = 100096
NPAD = NP - N
EP = 1638400
BD = 2048
B4 = 512
NSC = 2
NT = 16
RPT = NP // NT
DROW = 6272
DLAST = NP - 15 * DROW
TL = 8 * NP - 7

_mesh = lambda: plsc.VectorSubcoreMesh(core_axis_name="c", subcore_axis_name="s")
_sc_params = lambda: pltpu.CompilerParams(use_tc_tiling_on_sc=False)


def _fill1d(ref, n16, value):
    def body(i, _):
        ref[pl.ds(i * 16, 16)] = jnp.full((16,), value, jnp.float32)
        return 0
    lax.fori_loop(0, n16, body, 0)


def _deg_body(dst_h, z1_h, out0_h, out1_h, dstb_v, ones_v, accum):
    c = lax.axis_index("c")
    s = lax.axis_index("s")
    row0 = s * DROW
    _fill1d(ones_v, BD // 16, 1.0)

    @pl.when(s < 15)
    def _():
        pltpu.sync_copy(z1_h.at[pl.ds(row0, DROW)], accum.at[pl.ds(row0, DROW)])

    @pl.when(s == 15)
    def _():
        pltpu.sync_copy(z1_h.at[pl.ds(row0, DLAST)], accum.at[pl.ds(row0, DLAST)])

    plsc.subcore_barrier()
    e0 = (c * NT + s) * (EP // (NSC * NT))
    nb = EP // (NSC * NT) // BD

    def body(j, _):
        base = pl.multiple_of(e0 + j * BD, 128)
        pltpu.sync_copy(dst_h.at[pl.ds(base, BD)], dstb_v)
        pltpu.sync_copy(ones_v, accum.at[dstb_v], add=True)
        return 0

    lax.fori_loop(0, nb, body, 0)
    plsc.subcore_barrier()
    for cc, out_h in ((0, out0_h), (1, out1_h)):
        @pl.when(c == cc)
        def _(out_h=out_h):
            @pl.when(s < 15)
            def _():
                pltpu.sync_copy(accum.at[pl.ds(row0, DROW)],
                                out_h.at[pl.ds(row0, DROW)])

            @pl.when(s == 15)
            def _():
                pltpu.sync_copy(accum.at[pl.ds(row0, DLAST)],
                                out_h.at[pl.ds(row0, DLAST)])


def _sc_degree(dst, z1):
    return pl.kernel(
        _deg_body,
        out_type=(jax.ShapeDtypeStruct((NP,), jnp.float32),
                  jax.ShapeDtypeStruct((NP,), jnp.float32)),
        mesh=_mesh(),
        compiler_params=_sc_params(),
        scratch_types=[
            pltpu.VMEM((BD,), jnp.int32),
            pltpu.VMEM((BD,), jnp.float32),
            pltpu.VMEM_SHARED((NP,), jnp.float32),
        ],
    )(dst, z1)


def _agg_pass(ed_h, tview, out_h, out_col, eb, rows, accum, sem_g, sem_s,
              e0, nb):
    s = lax.axis_index("s")
    row0 = s * RPT

    def zf(i, _):
        rows[2, i] = jnp.zeros((16,), jnp.float32)
        eb[2, 1, pl.ds((i % (B4 // 16)) * 16, 16)] = jnp.zeros((16,),
                                                              jnp.int32)
        return 0

    lax.fori_loop(0, B4, zf, 0)
    _NF = RPT // B4
    _TAIL = RPT - _NF * B4
    for r in range(_NF):
        pltpu.sync_copy(rows.at[2], accum.at[pl.ds(row0 + r * B4, B4)])
    if _TAIL:
        pltpu.sync_copy(rows.at[2].at[pl.ds(0, _TAIL)],
                        accum.at[pl.ds(row0 + _NF * B4, _TAIL)])
    plsc.subcore_barrier()

    def body(jj, _):
        p = jj % 3
        pm = (jj + 2) % 3

        @pl.when(jj >= 2)
        def _():
            pltpu.make_async_copy(rows.at[p], accum.at[eb.at[p, 1]],
                                  sem_s).wait()

        jc = jnp.minimum(jj, nb - 1)
        base = pl.multiple_of(e0 + jc * B4, 128)
        pltpu.sync_copy(ed_h.at[:, pl.ds(base, B4)], eb.at[p])
        gd = pltpu.async_copy(tview.at[eb.at[p, 0]], rows.at[p], sem_g)
        pltpu.async_copy(rows.at[pm], accum.at[eb.at[pm, 1]], sem_s,
                         add=True)
        gd.wait()
        return 0

    lax.fori_loop(0, nb + 1, body, 0)
    for _ in range(2):
        pltpu.make_async_copy(rows.at[0], accum.at[eb.at[0, 1]],
                              sem_s).wait()
    plsc.subcore_barrier()
    _NF = RPT // B4
    _TAIL = RPT - _NF * B4
    for r in range(_NF):
        pltpu.sync_copy(accum.at[pl.ds(row0 + r * B4, B4)], rows.at[0])
        pltpu.sync_copy(rows.at[0],
                        out_h.at[pl.ds(row0 + r * B4, B4),
                                 pl.ds(out_col, 16)])
    if _TAIL:
        pltpu.sync_copy(accum.at[pl.ds(row0 + _NF * B4, _TAIL)],
                        rows.at[0].at[pl.ds(0, _TAIL)])
        pltpu.sync_copy(rows.at[0].at[pl.ds(0, _TAIL)],
                        out_h.at[pl.ds(row0 + _NF * B4, _TAIL),
                                 pl.ds(out_col, 16)])


def _agg1_body(ed_h, tflat_h, out_h, eb, rows, accum, sem_g, sem_s):
    c = lax.axis_index("c")
    s = lax.axis_index("s")
    ept = EP // (NSC * NT)
    _agg_pass(ed_h, tflat_h.at[pl.ds(0, TL)], out_h, 16 * c, eb,
              rows, accum, sem_g, sem_s, (c * NT + s) * ept, ept // B4)


def _agg2_body(ed_h, tflat_h, out_h, eb, rows, accum, sem_g, sem_s):
    c = lax.axis_index("c")
    s = lax.axis_index("s")
    ept = EP // NT
    e0 = s * ept

    def chunk(k, _):
        p = 2 * c + k
        _agg_pass(ed_h, tflat_h.at[pl.ds(p, TL)], out_h, 16 * p,
                  eb, rows, accum, sem_g, sem_s, e0, ept // B4)
        return 0

    lax.fori_loop(0, 2, chunk, 0)


def _sc_agg(body, ed, tflat):
    return pl.kernel(
        body,
        out_type=jax.ShapeDtypeStruct((NP, 128), jnp.float32),
        mesh=_mesh(),
        compiler_params=_sc_params(),
        scratch_types=[
            pltpu.VMEM((3, 2, B4), jnp.int32),
            pltpu.VMEM((3, B4, 16), jnp.float32),
            pltpu.VMEM_SHARED((NP, 16), jnp.float32),
            pltpu.SemaphoreType.DMA,
            pltpu.SemaphoreType.DMA,
        ],
    )(ed, tflat)


_BN = 1088
_BNC = 1000


def _tcB_body(u1p, hs1p, gf, W1, b1, out_o):
    dinv = hs1p[:, 8:9]
    agg5 = dinv * (u1p[:, 0:5] + u1p[:, 16:21] + hs1p[:, 0:5])
    h1 = jax.nn.relu(
        jnp.dot(agg5, W1[...], preferred_element_type=jnp.float32) + b1[...])
    hs2 = dinv * h1
    i = pl.program_id(0)
    rows = i * _BN + lax.broadcasted_iota(jnp.int32, (_BN, 1), 0)
    hs2 = jnp.where(rows < N, hs2, 0.0)
    out_o[...] = jnp.concatenate(
        [hs2, dinv, gf[...], jnp.zeros((_BN, 60), jnp.float32)], axis=1)


def _tcB(u1p, hs1p, gfp, W1, b1):
    return pl.pallas_call(
        _tcB_body,
        grid=(NP // _BN,),
        in_specs=[
            pl.BlockSpec((_BN, 128), lambda i: (i, 0)),
            pl.BlockSpec((_BN, 128), lambda i: (i, 0)),
            pl.BlockSpec((_BN, 3), lambda i: (i, 0)),
            pl.BlockSpec((5, 64), lambda i: (0, 0)),
            pl.BlockSpec((64,), lambda i: (0,)),
        ],
        out_specs=pl.BlockSpec((_BN, 128), lambda i: (i, 0)),
        out_shape=jax.ShapeDtypeStruct((NP, 128), jnp.float32),
    )(u1p, hs1p, gfp, W1, b1)


def _tcC_body(u2p, hs2p, W2, b2, Wf1, bf1, Wf2, bf2, Wo, bo, out_o):
    dinv = hs2p[:, 64:65]
    gf = hs2p[:, 65:68]
    agg = dinv * (u2p[:, :64] + hs2p[:, :64])
    h2v = jax.nn.relu(
        jnp.dot(agg, W2[...], preferred_element_type=jnp.float32) + b2[...])
    t = jax.nn.relu(
        jnp.dot(h2v, Wf1[:64, :], preferred_element_type=jnp.float32)
        + jnp.dot(gf, Wf1[64:67, :], preferred_element_type=jnp.float32)
        + bf1[...])
    t = jax.nn.relu(
        jnp.dot(t, Wf2[...], preferred_element_type=jnp.float32) + bf2[...])
    out_o[...] = (
        jnp.dot(t, Wo[...], preferred_element_type=jnp.float32) + bo[...])


def _tcC(u2p, hs2p, W2, b2, Wf1, bf1, Wf2, bf2, Wo, bo):
    return pl.pallas_call(
        _tcC_body,
        grid=(N // _BNC,),
        in_specs=[
            pl.BlockSpec((_BNC, 128), lambda i: (i, 0)),
            pl.BlockSpec((_BNC, 128), lambda i: (i, 0)),
            pl.BlockSpec((64, 64), lambda i: (0, 0)),
            pl.BlockSpec((64,), lambda i: (0,)),
            pl.BlockSpec((67, 64), lambda i: (0, 0)),
            pl.BlockSpec((64,), lambda i: (0,)),
            pl.BlockSpec((64, 64), lambda i: (0, 0)),
            pl.BlockSpec((64,), lambda i: (0,)),
            pl.BlockSpec((64, 30), lambda i: (0, 0)),
            pl.BlockSpec((30,), lambda i: (0,)),
        ],
        out_specs=pl.BlockSpec((_BNC, 30), lambda i: (i, 0)),
        out_shape=jax.ShapeDtypeStruct((N, 30), jnp.float32),
    )(u2p, hs2p, W2, b2, Wf1, bf1, Wf2, bf2, Wo, bo)


def kernel(x, edge_index, globf, W1, b1, W2, b2, Wf1, bf1, Wf2, bf2, Wo, bo):
    ei = edge_index.astype(jnp.int32)
    pad_tgt = N + (jnp.arange(EP - E, dtype=jnp.int32) % NPAD)
    src = jnp.concatenate([ei[0], pad_tgt])
    dst = jnp.concatenate([ei[1], pad_tgt])
    ed = jnp.stack([src * 8, dst])
    z1 = jnp.zeros((NP,), jnp.float32)

    dg0, dg1 = _sc_degree(dst, z1)
    dinv = lax.rsqrt(dg0 + dg1 + 1.0)
    x5 = jnp.pad(x, ((0, NPAD), (0, 0)))
    hs1p = jnp.concatenate(
        [dinv[:, None] * x5, jnp.zeros((NP, 3), jnp.float32),
         dinv[:, None], jnp.zeros((NP, 119), jnp.float32)], axis=1)

    u1p = _sc_agg(_agg1_body, ed, hs1p.reshape(8 * NP, 16))
    gfp = jnp.pad(globf, ((0, NPAD), (0, 0)))
    hs2p = _tcB(u1p, hs1p, gfp, W1, b1)

    u2p = _sc_agg(_agg2_body, ed, hs2p.reshape(8 * NP, 16))
    return _tcC(u2p, hs2p, W2, b2, Wf1, bf1, Wf2, bf2, Wo, bo)
